# baseline (speedup 1.0000x reference)
import functools

import jax
import jax.numpy as jnp
from jax.experimental import pallas as pl
from jax.experimental.pallas import tpu as pltpu

_NUM_CODES = 512
_CODE_DIM = 32
_COMMITMENT_COST = 0.25
_ROWS_PER_BLOCK = 8


def _vq_block_kernel(z_ref, cb_ref, zq_ref, idx_ref, loss_ref):
    bt = z_ref.shape[0] * z_ref.shape[1]
    z = z_ref[...].reshape(bt, _CODE_DIM)
    cb = cb_ref[...]

    r32 = jax.lax.broadcasted_iota(jnp.int32, (_CODE_DIM, _CODE_DIM), 0)
    c32 = jax.lax.broadcasted_iota(jnp.int32, (_CODE_DIM, _CODE_DIM), 1)
    eye = (r32 == c32).astype(jnp.float32)
    zsq_t = jax.lax.dot_general(
        eye, z * z, (((1,), (1,)), ((), ())),
        preferred_element_type=jnp.float32)
    acc = zsq_t
    w = _CODE_DIM
    while w > 1:
        w //= 2
        acc = acc[:w, :] + acc[w:2 * w, :]
    z2 = acc

    c2 = jnp.sum(cb * cb, axis=1, keepdims=True)
    dots_t = jax.lax.dot_general(
        cb, z, (((1,), (1,)), ((), ())),
        preferred_element_type=jnp.float32)
    dist_t = (z2 + c2) - 2.0 * dots_t

    m = jnp.min(dist_t, axis=0, keepdims=True)
    iota_f = jax.lax.broadcasted_iota(
        jnp.int32, dist_t.shape, 0).astype(jnp.float32)
    idx_f = jnp.min(jnp.where(dist_t == m, iota_f,
                              float(_NUM_CODES)), axis=0)
    idx = idx_f.astype(jnp.int32)

    onehot_t = (iota_f == idx_f[None, :]).astype(jnp.float32)
    zq = jax.lax.dot_general(
        onehot_t, cb, (((0,), (0,)), ((), ())),
        preferred_element_type=jnp.float32)

    zq_ref[...] = zq.reshape(zq_ref.shape)
    idx_ref[...] = idx.reshape(idx_ref.shape)

    loss_ref[0, 0, 0] = jnp.sum(m)


@functools.partial(jax.jit, static_argnames=())
def kernel(z, codebook):
    n_rows, row_len, d = z.shape
    n_tok = n_rows * row_len
    bt = _ROWS_PER_BLOCK * row_len
    grid = n_rows // _ROWS_PER_BLOCK
    zq, idx, loss_sum = pl.pallas_call(
        _vq_block_kernel,
        grid=(grid,),
        in_specs=[
            pl.BlockSpec((_ROWS_PER_BLOCK, row_len, d), lambda i: (i, 0, 0)),
            pl.BlockSpec((_NUM_CODES, _CODE_DIM), lambda i: (0, 0)),
        ],
        out_specs=[
            pl.BlockSpec((_ROWS_PER_BLOCK, row_len, d), lambda i: (i, 0, 0)),
            pl.BlockSpec((_ROWS_PER_BLOCK, 1, row_len), lambda i: (i, 0, 0)),
            pl.BlockSpec((1, 1, 1), lambda i: (i, 0, 0), memory_space=pltpu.SMEM),
        ],
        out_shape=[
            jax.ShapeDtypeStruct(z.shape, jnp.float32),
            jax.ShapeDtypeStruct((n_rows, 1, row_len), jnp.int32),
            jax.ShapeDtypeStruct((grid, 1, 1), jnp.float32),
        ],
        compiler_params=pltpu.CompilerParams(
            dimension_semantics=("parallel",)),
    )(z, codebook)
    loss = (_COMMITMENT_COST / (n_tok * d)) * jnp.sum(loss_sum)
    return (zq, loss, idx.reshape(n_rows, row_len))


# speedup vs baseline: 1.9546x; 1.9546x over previous
import functools

import jax
import jax.numpy as jnp
from jax.experimental import pallas as pl
from jax.experimental.pallas import tpu as pltpu

_NUM_CODES = 512
_CODE_DIM = 32
_COMMITMENT_COST = 0.25
_ROWS_PER_BLOCK = 8


def _vq_block_kernel(z_ref, cb_ref, zq_ref, idx_ref, loss_ref):
    i = pl.program_id(0)
    n_r, _, row_len = z_ref.shape
    cb = cb_ref[...]
    c2 = jnp.sum(cb * cb, axis=1, keepdims=True)
    iota_f = jax.lax.broadcasted_iota(
        jnp.int32, (_NUM_CODES, row_len), 0).astype(jnp.float32)

    part = jnp.zeros((), jnp.float32)
    for r in range(n_r):
        zr = z_ref[r]
        acc = zr * zr
        w = _CODE_DIM
        while w > 1:
            w //= 2
            acc = acc[:w, :] + acc[w:2 * w, :]
        dots = jax.lax.dot_general(
            cb, zr, (((1,), (0,)), ((), ())),
            preferred_element_type=jnp.float32)
        dist = (acc + c2) - 2.0 * dots

        m = jnp.min(dist, axis=0, keepdims=True)
        idx_f = jnp.min(jnp.where(dist == m, iota_f,
                                  float(_NUM_CODES)), axis=0)
        idx_ref[r, :] = idx_f.astype(jnp.int32)

        onehot_t = (iota_f == idx_f[None, :]).astype(jnp.float32)
        zq_t = jax.lax.dot_general(
            cb, onehot_t, (((0,), (0,)), ((), ())),
            preferred_element_type=jnp.float32)
        zq_ref[r] = zq_t

        part = part + jnp.sum(m)

    @pl.when(i == 0)
    def _init():
        loss_ref[0, 0] = part

    @pl.when(i != 0)
    def _acc():
        loss_ref[0, 0] = loss_ref[0, 0] + part


@functools.partial(jax.jit, static_argnames=())
def kernel(z, codebook):
    n_rows, row_len, d = z.shape
    n_tok = n_rows * row_len
    grid = n_rows // _ROWS_PER_BLOCK
    z_phys = jnp.transpose(z, (0, 2, 1))
    zq_phys, idx, loss_sum = pl.pallas_call(
        _vq_block_kernel,
        grid=(grid,),
        in_specs=[
            pl.BlockSpec((_ROWS_PER_BLOCK, d, row_len), lambda i: (i, 0, 0)),
            pl.BlockSpec((_NUM_CODES, _CODE_DIM), lambda i: (0, 0)),
        ],
        out_specs=[
            pl.BlockSpec((_ROWS_PER_BLOCK, d, row_len), lambda i: (i, 0, 0)),
            pl.BlockSpec((_ROWS_PER_BLOCK, row_len), lambda i: (i, 0)),
            pl.BlockSpec(memory_space=pltpu.SMEM),
        ],
        out_shape=[
            jax.ShapeDtypeStruct((n_rows, d, row_len), jnp.float32),
            jax.ShapeDtypeStruct((n_rows, row_len), jnp.int32),
            jax.ShapeDtypeStruct((1, 1), jnp.float32),
        ],
    )(z_phys, codebook)
    zq = jnp.transpose(zq_phys, (0, 2, 1))
    loss = (_COMMITMENT_COST / (n_tok * d)) * loss_sum[0, 0]
    return (zq, loss, idx)


# speedup vs baseline: 1.9637x; 1.0047x over previous
import functools

import jax
import jax.numpy as jnp
from jax.experimental import pallas as pl
from jax.experimental.pallas import tpu as pltpu

_NUM_CODES = 512
_CODE_DIM = 32
_COMMITMENT_COST = 0.25
_ROWS_PER_BLOCK = 16


def _vq_block_kernel(z_ref, cb_ref, zq_ref, idx_ref, loss_ref):
    i = pl.program_id(0)
    n_r, _, row_len = z_ref.shape
    cb = cb_ref[...]
    c2 = jnp.sum(cb * cb, axis=1, keepdims=True)
    iota_f = jax.lax.broadcasted_iota(
        jnp.int32, (_NUM_CODES, row_len), 0).astype(jnp.float32)

    part = jnp.zeros((), jnp.float32)
    for r in range(n_r):
        zr = z_ref[r]
        acc = zr * zr
        w = _CODE_DIM
        while w > 1:
            w //= 2
            acc = acc[:w, :] + acc[w:2 * w, :]
        dots = jax.lax.dot_general(
            cb, zr, (((1,), (0,)), ((), ())),
            preferred_element_type=jnp.float32)
        dist = (acc + c2) - 2.0 * dots

        m = jnp.min(dist, axis=0, keepdims=True)
        idx_f = jnp.min(jnp.where(dist == m, iota_f,
                                  float(_NUM_CODES)), axis=0)
        idx_ref[r, :] = idx_f.astype(jnp.int32)

        onehot_t = (iota_f == idx_f[None, :]).astype(jnp.float32)
        zq_t = jax.lax.dot_general(
            cb, onehot_t, (((0,), (0,)), ((), ())),
            preferred_element_type=jnp.float32)
        zq_ref[r] = zq_t

        part = part + jnp.sum(m)

    @pl.when(i == 0)
    def _init():
        loss_ref[0, 0] = part

    @pl.when(i != 0)
    def _acc():
        loss_ref[0, 0] = loss_ref[0, 0] + part


@functools.partial(jax.jit, static_argnames=())
def kernel(z, codebook):
    n_rows, row_len, d = z.shape
    n_tok = n_rows * row_len
    grid = n_rows // _ROWS_PER_BLOCK
    z_phys = jnp.transpose(z, (0, 2, 1))
    zq_phys, idx, loss_sum = pl.pallas_call(
        _vq_block_kernel,
        grid=(grid,),
        in_specs=[
            pl.BlockSpec((_ROWS_PER_BLOCK, d, row_len), lambda i: (i, 0, 0)),
            pl.BlockSpec((_NUM_CODES, _CODE_DIM), lambda i: (0, 0)),
        ],
        out_specs=[
            pl.BlockSpec((_ROWS_PER_BLOCK, d, row_len), lambda i: (i, 0, 0)),
            pl.BlockSpec((_ROWS_PER_BLOCK, row_len), lambda i: (i, 0)),
            pl.BlockSpec(memory_space=pltpu.SMEM),
        ],
        out_shape=[
            jax.ShapeDtypeStruct((n_rows, d, row_len), jnp.float32),
            jax.ShapeDtypeStruct((n_rows, row_len), jnp.int32),
            jax.ShapeDtypeStruct((1, 1), jnp.float32),
        ],
    )(z_phys, codebook)
    zq = jnp.transpose(zq_phys, (0, 2, 1))
    loss = (_COMMITMENT_COST / (n_tok * d)) * loss_sum[0, 0]
    return (zq, loss, idx)
